# baseline (speedup 1.0000x reference)
import functools

import numpy as np
import jax
import jax.numpy as jnp
from jax import lax
from jax.experimental import pallas as pl
from jax.experimental.pallas import tpu as pltpu
from jax.experimental.pallas import tpu_sc as plsc

_FIELD_DIMS = [1000] * 26
_F = 26
_FEAT = sum(_FIELD_DIMS)
_D = 16
_PAIRS = _F * (_F - 1) // 2
_INTER = _PAIRS * _D
_HROWS = 41
_B = 4096
_OFFS = np.asarray([0, *np.cumsum(_FIELD_DIMS)[:-1]], dtype=np.int32)

_NW = 32
_BPW = _B // _NW

_NB = 32
_NA = 80
_NR = _NB + _NA


def _pair_perm():
    def old_idx(f, g):
        f, g = min(f, g), max(f, g)
        return f * (2 * _F - f - 1) // 2 + (g - f - 1)
    perm = []
    for r in range(_F - 1):
        for m in range(13):
            if m == 0:
                a, b = r, _F - 1
            else:
                a = (r + m) % (_F - 1)
                b = (r - m) % (_F - 1)
            perm.append(old_idx(a, b))
    assert sorted(perm) == list(range(_PAIRS))
    return np.asarray(perm, dtype=np.int32)


_PERM = _pair_perm()


def _sc_make():
    mesh = plsc.VectorSubcoreMesh(core_axis_name="c", subcore_axis_name="s")

    @functools.partial(
        pl.kernel,
        mesh=mesh,
        compiler_params=pltpu.CompilerParams(
            use_tc_tiling_on_sc=False, needs_layout_passes=False),
        out_type=[
            jax.ShapeDtypeStruct((_B * _HROWS, 128), jnp.float32),
        ],
        scratch_types=[
            pltpu.VMEM((_BPW, 128), jnp.int32),
            pltpu.VMEM((2, _NA), jnp.int32),
            pltpu.VMEM((_NA,), jnp.int32),
            pltpu.VMEM((_NA,), jnp.int32),
            pltpu.VMEM((2, _NR, 128), jnp.float32),
            pltpu.VMEM((2, _HROWS, 128), jnp.float32),
            pltpu.SMEM((_PAIRS,), jnp.int32),
            pltpu.SMEM((_PAIRS,), jnp.int32),
            pltpu.SemaphoreType.DMA,
            pltpu.SemaphoreType.DMA,
            pltpu.SemaphoreType.DMA,
            pltpu.SemaphoreType.DMA,
        ],
    )
    def sc_interactions(ta_hbm, tb_hbm, xpad_hbm, h_hbm,
                        idx_v, ixa_v, ga_tab, oa_tab, R_x, h_v,
                        addr_a, addr_b,
                        gsem0, gsem1, wsem0, wsem1):
        wid = lax.axis_index("s") * 2 + lax.axis_index("c")
        base = wid * _BPW

        pltpu.sync_copy(xpad_hbm.at[pl.ds(base, _BPW)], idx_v)

        lane_iota = lax.iota(jnp.int32, 16)

        def ga_chunk(k, _):
            vj = lane_iota + k * 16
            vphi = jnp.minimum(lax.div(vj, jnp.int32(_F)), 2)
            vg = jnp.minimum(vj - vphi * _F, _F - 1)
            ga_tab[pl.ds(pl.multiple_of(k * 16, 16), 16)] = vg
            oa_tab[pl.ds(pl.multiple_of(k * 16, 16), 16)] = vphi * _FEAT
            return 0
        lax.fori_loop(0, _NA // 16, ga_chunk, 0)

        def addr_of(f, g):
            is_b = f >= 24
            row = jnp.where(is_b, g,
                            _NB + lax.shift_right_logical(f, 3) * _F + g)
            lane = jnp.where(is_b, (f - 24) * _D,
                             lax.bitwise_and(f, 7) * _D)
            return row * 128 + lane

        def addr_fill(r, _):
            for m in range(13):
                if m == 0:
                    a, bb = r, _F - 1
                else:
                    a = r + m
                    a = jnp.where(a >= _F - 1, a - (_F - 1), a)
                    bb = r - m + (_F - 1)
                    bb = jnp.where(bb >= _F - 1, bb - (_F - 1), bb)
                addr_a[r * 13 + m] = addr_of(a, bb)
                addr_b[r * 13 + m] = addr_of(bb, a)
            return 0
        lax.fori_loop(0, _F - 1, addr_fill, 0)

        zero16 = jnp.zeros((16,), jnp.float32)
        for slot in (0, 1):
            for j in range(3):
                h_v[slot, _HROWS - 1, pl.ds(80 + j * 16, 16)] = zero16

        def build_ixa(b, slot):
            bvec = jnp.zeros((16,), jnp.int32) + b
            def chunk(k, _):
                sl = pl.ds(pl.multiple_of(k * 16, 16), 16)
                vg = ga_tab[sl]
                xv = plsc.load_gather(idx_v, [bvec, vg])
                ixa_v[slot, sl] = xv + oa_tab[sl]
                return 0
            lax.fori_loop(0, _NA // 16, chunk, 0)

        def fire_gather(b, slot, gsem):
            pltpu.async_copy(
                tb_hbm.at[idx_v.at[b, pl.ds(0, _NB)]],
                R_x.at[slot, pl.ds(0, _NB)], gsem)
            pltpu.async_copy(
                ta_hbm.at[ixa_v.at[slot]],
                R_x.at[slot, pl.ds(_NB, _NA)], gsem)

        def wait_gather(b, slot, gsem):
            pltpu.make_async_copy(
                tb_hbm.at[idx_v.at[b, pl.ds(0, _NB)]],
                R_x.at[slot, pl.ds(0, _NB)], gsem).wait()
            pltpu.make_async_copy(
                ta_hbm.at[ixa_v.at[slot]],
                R_x.at[slot, pl.ds(_NB, _NA)], gsem).wait()

        build_ixa(0, 0)
        fire_gather(0, 0, gsem0)
        build_ixa(1, 1)

        def half_step(b, slot, o_slot, gsem, o_gsem, wsem):
            wait_gather(b, slot, gsem)
            @pl.when(b + 1 < _BPW)
            def _():
                fire_gather(b + 1, o_slot, o_gsem)
            @pl.when(b + 2 < _BPW)
            def _():
                build_ixa(b + 2, slot)
            @pl.when(b >= 2)
            def _():
                pltpu.make_async_copy(
                    h_v.at[slot],
                    h_hbm.at[pl.ds((base + b - 2) * _HROWS, _HROWS)],
                    wsem).wait()

            def round_loop(r, _):
                p0 = r * 13
                for m in range(13):
                    p = p0 + m
                    aa = addr_a[p]
                    ab = addr_b[p]
                    va = R_x[slot, lax.shift_right_logical(aa, 7),
                             pl.ds(pl.multiple_of(
                                 lax.bitwise_and(aa, 127), 16), 16)]
                    vb = R_x[slot, lax.shift_right_logical(ab, 7),
                             pl.ds(pl.multiple_of(
                                 lax.bitwise_and(ab, 127), 16), 16)]
                    q = p * 16
                    h_v[slot, q >> 7,
                        pl.ds(pl.multiple_of(q & 127, 16), 16)] = va * vb
                return 0
            lax.fori_loop(0, _F - 1, round_loop, 0)

            def w_loop(g, acc):
                return acc + R_x[slot, g, pl.ds(32, 16)]
            h_v[slot, _HROWS - 1, pl.ds(80, 16)] = lax.fori_loop(
                0, _F, w_loop, jnp.zeros((16,), jnp.float32))

            pltpu.async_copy(
                h_v.at[slot],
                h_hbm.at[pl.ds((base + b) * _HROWS, _HROWS)], wsem)

        def iter_body(i, _):
            b0 = 2 * i
            half_step(b0, 0, 1, gsem0, gsem1, wsem0)
            half_step(b0 + 1, 1, 0, gsem1, gsem0, wsem1)
            return 0
        lax.fori_loop(0, _BPW // 2, iter_body, 0)

        pltpu.make_async_copy(
            h_v.at[0],
            h_hbm.at[pl.ds((base + _BPW - 2) * _HROWS, _HROWS)], wsem0).wait()
        pltpu.make_async_copy(
            h_v.at[1],
            h_hbm.at[pl.ds((base + _BPW - 1) * _HROWS, _HROWS)], wsem1).wait()

    return sc_interactions


_sc_interactions = _sc_make()


def _prep_a_body(emb_ref, out_ref):
    out_ref[...] = jnp.concatenate(
        [emb_ref[f] for f in range(8)], axis=-1)


_prep_a = pl.pallas_call(
    _prep_a_body,
    grid=(3, 26),
    in_specs=[pl.BlockSpec((8, 1000, _D), lambda p, i: (p, i, 0))],
    out_specs=pl.BlockSpec((1000, 128), lambda p, i: (p * 26 + i, 0)),
    out_shape=jax.ShapeDtypeStruct((3 * _FEAT, 128), jnp.float32),
)


def _prep_b_body(emb_ref, w_ref, out_ref):
    out_ref[...] = jnp.concatenate(
        [emb_ref[0], emb_ref[1], w_ref[...],
         jnp.zeros((1000, 128 - 2 * _D - 16), jnp.float32)], axis=-1)


_prep_b = pl.pallas_call(
    _prep_b_body,
    grid=(26,),
    in_specs=[
        pl.BlockSpec((2, 1000, _D), lambda i: (12, i, 0)),
        pl.BlockSpec((1000, 16), lambda i: (i, 0)),
    ],
    out_specs=pl.BlockSpec((1000, 128), lambda i: (i, 0)),
    out_shape=jax.ShapeDtypeStruct((_FEAT, 128), jnp.float32),
)

_BT = 512


def _mlp_body(h_ref, W1_ref, b1_ref, W2_ref, b2_ref, W3_ref, b3_ref, e_ref,
              out_ref):
    h2 = h_ref[...].reshape(_BT, _HROWS * 128)
    a1 = jnp.dot(h2, W1_ref[...], preferred_element_type=jnp.float32)
    a1 = jnp.maximum(a1 + b1_ref[...], 0.0)
    a2 = jnp.dot(a1, W2_ref[...], preferred_element_type=jnp.float32)
    a2 = jnp.maximum(a2 + b2_ref[...], 0.0)
    a3 = jnp.dot(a2, W3_ref[...], preferred_element_type=jnp.float32)
    fo = jnp.dot(h2, e_ref[...], preferred_element_type=jnp.float32)
    out_ref[...] = a3 + fo + b3_ref[...]


_mlp_call = pl.pallas_call(
    _mlp_body,
    grid=(_B // _BT,),
    in_specs=[
        pl.BlockSpec((_BT * _HROWS, 128), lambda i: (i, 0)),
        pl.BlockSpec((_HROWS * 128, 64), lambda i: (0, 0)),
        pl.BlockSpec((1, 64), lambda i: (0, 0)),
        pl.BlockSpec((64, 32), lambda i: (0, 0)),
        pl.BlockSpec((1, 32), lambda i: (0, 0)),
        pl.BlockSpec((32, 1), lambda i: (0, 0)),
        pl.BlockSpec((1, 1), lambda i: (0, 0)),
        pl.BlockSpec((_HROWS * 128, 1), lambda i: (0, 0)),
    ],
    out_specs=pl.BlockSpec((_BT, 1), lambda i: (i, 0)),
    out_shape=jax.ShapeDtypeStruct((_B, 1), jnp.float32),
)


def kernel(x, emb, w_lin, b_lin, W1, b1, W2, b2, W3, b3):
    x_off = x + jnp.asarray(_OFFS)[None, :]
    x_pad = jnp.pad(x_off, ((0, 0), (0, 128 - _F)))
    w16 = jnp.pad(w_lin.reshape(_FEAT, 1), ((0, 0), (0, 15)))
    ta = _prep_a(emb)
    tb = _prep_b(emb, w16)
    (h,) = _sc_interactions(ta, tb, x_pad)
    W1perm = W1.reshape(_PAIRS, _D, 64)[_PERM].reshape(_INTER, 64)
    W1p = jnp.concatenate(
        [W1perm, jnp.zeros((_HROWS * 128 - _INTER, 64), jnp.float32)], axis=0)
    e = jnp.zeros((_HROWS * 128, 1), jnp.float32).at[_INTER, 0].set(1.0)
    out = _mlp_call(h, W1p, b1.reshape(1, 64), W2, b2.reshape(1, 32),
                    W3, (b3 + b_lin).reshape(1, 1), e)
    return out[:, 0]


# speedup vs baseline: 1.1722x; 1.1722x over previous
import functools

import numpy as np
import jax
import jax.numpy as jnp
from jax import lax
from jax.experimental import pallas as pl
from jax.experimental.pallas import tpu as pltpu
from jax.experimental.pallas import tpu_sc as plsc

_FIELD_DIMS = [1000] * 26
_F = 26
_FEAT = sum(_FIELD_DIMS)
_D = 16
_PAIRS = _F * (_F - 1) // 2
_INTER = _PAIRS * _D
_HROWS = 41
_B = 4096
_OFFS = np.asarray([0, *np.cumsum(_FIELD_DIMS)[:-1]], dtype=np.int32)

_NW = 32
_BPW = _B // _NW

_NB = 32
_NA = 80
_NR = _NB + _NA


def _pair_perm():
    def old_idx(f, g):
        f, g = min(f, g), max(f, g)
        return f * (2 * _F - f - 1) // 2 + (g - f - 1)
    perm = []
    for r in range(_F - 1):
        for m in range(13):
            if m == 0:
                a, b = r, _F - 1
            else:
                a = (r + m) % (_F - 1)
                b = (r - m) % (_F - 1)
            perm.append(old_idx(a, b))
    assert sorted(perm) == list(range(_PAIRS))
    return np.asarray(perm, dtype=np.int32)


_PERM = _pair_perm()


def _sc_make():
    mesh = plsc.VectorSubcoreMesh(core_axis_name="c", subcore_axis_name="s")

    @functools.partial(
        pl.kernel,
        mesh=mesh,
        compiler_params=pltpu.CompilerParams(
            use_tc_tiling_on_sc=False, needs_layout_passes=False),
        out_type=[
            jax.ShapeDtypeStruct((_B * _HROWS, 128), jnp.float32),
        ],
        scratch_types=[
            pltpu.VMEM((_BPW, 128), jnp.int32),
            pltpu.VMEM((2, _NA), jnp.int32),
            pltpu.VMEM((_NA,), jnp.int32),
            pltpu.VMEM((_NA,), jnp.int32),
            pltpu.VMEM((2, _NR, 128), jnp.float32),
            pltpu.VMEM((2, _HROWS, 128), jnp.float32),
            pltpu.SemaphoreType.DMA,
            pltpu.SemaphoreType.DMA,
            pltpu.SemaphoreType.DMA,
            pltpu.SemaphoreType.DMA,
        ],
    )
    def sc_interactions(ta_hbm, tb_hbm, xpad_hbm, h_hbm,
                        idx_v, ixa_v, ga_tab, oa_tab, R_x, h_v,
                        gsem0, gsem1, wsem0, wsem1):
        wid = lax.axis_index("s") * 2 + lax.axis_index("c")
        base = wid * _BPW

        pltpu.sync_copy(xpad_hbm.at[pl.ds(base, _BPW)], idx_v)

        lane_iota = lax.iota(jnp.int32, 16)

        def ga_chunk(k, _):
            vj = lane_iota + k * 16
            vphi = jnp.minimum(lax.div(vj, jnp.int32(_F)), 2)
            vg = jnp.minimum(vj - vphi * _F, _F - 1)
            ga_tab[pl.ds(pl.multiple_of(k * 16, 16), 16)] = vg
            oa_tab[pl.ds(pl.multiple_of(k * 16, 16), 16)] = vphi * _FEAT
            return 0
        lax.fori_loop(0, _NA // 16, ga_chunk, 0)

        zero16 = jnp.zeros((16,), jnp.float32)
        for slot in (0, 1):
            for j in range(3):
                h_v[slot, _HROWS - 1, pl.ds(80 + j * 16, 16)] = zero16

        def build_ixa(b, slot):
            bvec = jnp.zeros((16,), jnp.int32) + b
            def chunk(k, _):
                sl = pl.ds(pl.multiple_of(k * 16, 16), 16)
                vg = ga_tab[sl]
                xv = plsc.load_gather(idx_v, [bvec, vg])
                ixa_v[slot, sl] = xv + oa_tab[sl]
                return 0
            lax.fori_loop(0, _NA // 16, chunk, 0)

        def fire_gather(b, slot, gsem):
            pltpu.async_copy(
                tb_hbm.at[idx_v.at[b, pl.ds(0, _NB)]],
                R_x.at[slot, pl.ds(0, _NB)], gsem)
            pltpu.async_copy(
                ta_hbm.at[ixa_v.at[slot]],
                R_x.at[slot, pl.ds(_NB, _NA)], gsem)

        def wait_gather(b, slot, gsem):
            pltpu.make_async_copy(
                tb_hbm.at[idx_v.at[b, pl.ds(0, _NB)]],
                R_x.at[slot, pl.ds(0, _NB)], gsem).wait()
            pltpu.make_async_copy(
                ta_hbm.at[ixa_v.at[slot]],
                R_x.at[slot, pl.ds(_NB, _NA)], gsem).wait()

        build_ixa(0, 0)
        fire_gather(0, 0, gsem0)
        build_ixa(1, 1)

        def half_step(b, slot, o_slot, gsem, o_gsem, wsem):
            wait_gather(b, slot, gsem)
            @pl.when(b + 1 < _BPW)
            def _():
                fire_gather(b + 1, o_slot, o_gsem)
            @pl.when(b + 2 < _BPW)
            def _():
                build_ixa(b + 2, slot)
            @pl.when(b >= 2)
            def _():
                pltpu.make_async_copy(
                    h_v.at[slot],
                    h_hbm.at[pl.ds((base + b - 2) * _HROWS, _HROWS)],
                    wsem).wait()

            def round_loop(r, _):
                q0 = r * 208
                row_a = _NB + lax.shift_right_logical(r, 3) * _F + (_F - 1)
                lane_a = lax.bitwise_and(r, 7) * _D
                va = R_x[slot, row_a,
                         pl.ds(pl.multiple_of(lane_a, 16), 16)]
                vb = R_x[slot, r, pl.ds(_D, 16)]
                h_v[slot, q0 >> 7,
                    pl.ds(pl.multiple_of(q0 & 127, 16), 16)] = va * vb
                for m in range(1, 13):
                    a = r + m
                    a = jnp.where(a >= _F - 1, a - (_F - 1), a)
                    bb = r - m + (_F - 1)
                    bb = jnp.where(bb >= _F - 1, bb - (_F - 1), bb)
                    va = R_x[slot,
                             _NB + lax.shift_right_logical(a, 3) * _F + bb,
                             pl.ds(pl.multiple_of(
                                 lax.bitwise_and(a, 7) * _D, 16), 16)]
                    vb = R_x[slot,
                             _NB + lax.shift_right_logical(bb, 3) * _F + a,
                             pl.ds(pl.multiple_of(
                                 lax.bitwise_and(bb, 7) * _D, 16), 16)]
                    q = q0 + m * 16
                    h_v[slot, q >> 7,
                        pl.ds(pl.multiple_of(q & 127, 16), 16)] = va * vb
                return 0
            lax.fori_loop(0, _F - 1, round_loop, 0)

            def w_loop(g, acc):
                return acc + R_x[slot, g, pl.ds(32, 16)]
            h_v[slot, _HROWS - 1, pl.ds(80, 16)] = lax.fori_loop(
                0, _F, w_loop, jnp.zeros((16,), jnp.float32))

            pltpu.async_copy(
                h_v.at[slot],
                h_hbm.at[pl.ds((base + b) * _HROWS, _HROWS)], wsem)

        def iter_body(i, _):
            b0 = 2 * i
            half_step(b0, 0, 1, gsem0, gsem1, wsem0)
            half_step(b0 + 1, 1, 0, gsem1, gsem0, wsem1)
            return 0
        lax.fori_loop(0, _BPW // 2, iter_body, 0)

        pltpu.make_async_copy(
            h_v.at[0],
            h_hbm.at[pl.ds((base + _BPW - 2) * _HROWS, _HROWS)], wsem0).wait()
        pltpu.make_async_copy(
            h_v.at[1],
            h_hbm.at[pl.ds((base + _BPW - 1) * _HROWS, _HROWS)], wsem1).wait()

    return sc_interactions


_sc_interactions = _sc_make()


_BT = 512


def _mlp_body(h_ref, W1_ref, b1_ref, W2_ref, b2_ref, W3_ref, b3_ref, e_ref,
              out_ref):
    h2 = h_ref[...].reshape(_BT, _HROWS * 128)
    a1 = jnp.dot(h2, W1_ref[...], preferred_element_type=jnp.float32)
    a1 = jnp.maximum(a1 + b1_ref[...], 0.0)
    a2 = jnp.dot(a1, W2_ref[...], preferred_element_type=jnp.float32)
    a2 = jnp.maximum(a2 + b2_ref[...], 0.0)
    a3 = jnp.dot(a2, W3_ref[...], preferred_element_type=jnp.float32)
    fo = jnp.dot(h2, e_ref[...], preferred_element_type=jnp.float32)
    out_ref[...] = a3 + fo + b3_ref[...]


_mlp_call = pl.pallas_call(
    _mlp_body,
    grid=(_B // _BT,),
    in_specs=[
        pl.BlockSpec((_BT * _HROWS, 128), lambda i: (i, 0)),
        pl.BlockSpec((_HROWS * 128, 64), lambda i: (0, 0)),
        pl.BlockSpec((1, 64), lambda i: (0, 0)),
        pl.BlockSpec((64, 32), lambda i: (0, 0)),
        pl.BlockSpec((1, 32), lambda i: (0, 0)),
        pl.BlockSpec((32, 1), lambda i: (0, 0)),
        pl.BlockSpec((1, 1), lambda i: (0, 0)),
        pl.BlockSpec((_HROWS * 128, 1), lambda i: (0, 0)),
    ],
    out_specs=pl.BlockSpec((_BT, 1), lambda i: (i, 0)),
    out_shape=jax.ShapeDtypeStruct((_B, 1), jnp.float32),
)


def kernel(x, emb, w_lin, b_lin, W1, b1, W2, b2, W3, b3):
    x_off = x + jnp.asarray(_OFFS)[None, :]
    x_pad = jnp.pad(x_off, ((0, 0), (0, 128 - _F)))
    ta = jnp.concatenate(
        [jnp.transpose(emb[8 * p:8 * p + 8], (1, 0, 2)).reshape(_FEAT, 128)
         for p in range(3)], axis=0)
    tb = jnp.concatenate(
        [emb[24], emb[25], w_lin.reshape(_FEAT, 1),
         jnp.zeros((_FEAT, 128 - 2 * _D - 1), jnp.float32)], axis=1)
    (h,) = _sc_interactions(ta, tb, x_pad)
    W1perm = W1.reshape(_PAIRS, _D, 64)[_PERM].reshape(_INTER, 64)
    W1p = jnp.concatenate(
        [W1perm, jnp.zeros((_HROWS * 128 - _INTER, 64), jnp.float32)], axis=0)
    e = jnp.zeros((_HROWS * 128, 1), jnp.float32).at[_INTER, 0].set(1.0)
    out = _mlp_call(h, W1p, b1.reshape(1, 64), W2, b2.reshape(1, 32),
                    W3, (b3 + b_lin).reshape(1, 1), e)
    return out[:, 0]


# speedup vs baseline: 1.1724x; 1.0001x over previous
import functools

import numpy as np
import jax
import jax.numpy as jnp
from jax import lax
from jax.experimental import pallas as pl
from jax.experimental.pallas import tpu as pltpu
from jax.experimental.pallas import tpu_sc as plsc

_FIELD_DIMS = [1000] * 26
_F = 26
_FEAT = sum(_FIELD_DIMS)
_D = 16
_PAIRS = _F * (_F - 1) // 2
_INTER = _PAIRS * _D
_HROWS = 41
_B = 4096
_OFFS = np.asarray([0, *np.cumsum(_FIELD_DIMS)[:-1]], dtype=np.int32)

_NW = 32
_BPW = _B // _NW

_NB = 32
_NA = 80
_NR = _NB + _NA


def _pair_perm():
    def old_idx(f, g):
        f, g = min(f, g), max(f, g)
        return f * (2 * _F - f - 1) // 2 + (g - f - 1)
    perm = []
    for f in range(24):
        for g in range(f + 1, 24):
            perm.append(old_idx(f, g))
    for g in (24, 25):
        for f in range(24):
            perm.append(old_idx(f, g))
    perm.append(old_idx(24, 25))
    assert sorted(perm) == list(range(_PAIRS))
    return np.asarray(perm, dtype=np.int32)


_PERM = _pair_perm()


def _sc_make():
    mesh = plsc.VectorSubcoreMesh(core_axis_name="c", subcore_axis_name="s")

    @functools.partial(
        pl.kernel,
        mesh=mesh,
        compiler_params=pltpu.CompilerParams(
            use_tc_tiling_on_sc=False, needs_layout_passes=False),
        out_type=[
            jax.ShapeDtypeStruct((_B * _HROWS, 128), jnp.float32),
        ],
        scratch_types=[
            pltpu.VMEM((_BPW, 128), jnp.int32),
            pltpu.VMEM((2, _NA), jnp.int32),
            pltpu.VMEM((_NA,), jnp.int32),
            pltpu.VMEM((_NA,), jnp.int32),
            pltpu.VMEM((2, _NR, 128), jnp.float32),
            pltpu.VMEM((2, _HROWS, 128), jnp.float32),
            pltpu.SemaphoreType.DMA,
            pltpu.SemaphoreType.DMA,
            pltpu.SemaphoreType.DMA,
            pltpu.SemaphoreType.DMA,
        ],
    )
    def sc_interactions(ta_hbm, tb_hbm, xpad_hbm, h_hbm,
                        idx_v, ixa_v, ga_tab, oa_tab, R_x, h_v,
                        gsem0, gsem1, wsem0, wsem1):
        wid = lax.axis_index("s") * 2 + lax.axis_index("c")
        base = wid * _BPW

        pltpu.sync_copy(xpad_hbm.at[pl.ds(base, _BPW)], idx_v)

        lane_iota = lax.iota(jnp.int32, 16)

        def ga_chunk(k, _):
            vj = lane_iota + k * 16
            vphi = jnp.minimum(lax.div(vj, jnp.int32(_F)), 2)
            vg = jnp.minimum(vj - vphi * _F, _F - 1)
            ga_tab[pl.ds(pl.multiple_of(k * 16, 16), 16)] = vg
            oa_tab[pl.ds(pl.multiple_of(k * 16, 16), 16)] = vphi * _FEAT
            return 0
        lax.fori_loop(0, _NA // 16, ga_chunk, 0)

        zero16 = jnp.zeros((16,), jnp.float32)
        for slot in (0, 1):
            for j in range(3):
                h_v[slot, _HROWS - 1, pl.ds(80 + j * 16, 16)] = zero16

        def build_ixa(b, slot):
            bvec = jnp.zeros((16,), jnp.int32) + b
            def chunk(k, _):
                sl = pl.ds(pl.multiple_of(k * 16, 16), 16)
                vg = ga_tab[sl]
                xv = plsc.load_gather(idx_v, [bvec, vg])
                ixa_v[slot, sl] = xv + oa_tab[sl]
                return 0
            lax.fori_loop(0, _NA // 16, chunk, 0)

        def fire_gather(b, slot, gsem):
            pltpu.async_copy(
                tb_hbm.at[idx_v.at[b, pl.ds(0, _NB)]],
                R_x.at[slot, pl.ds(0, _NB)], gsem)
            pltpu.async_copy(
                ta_hbm.at[ixa_v.at[slot]],
                R_x.at[slot, pl.ds(_NB, _NA)], gsem)

        def wait_gather(b, slot, gsem):
            pltpu.make_async_copy(
                tb_hbm.at[idx_v.at[b, pl.ds(0, _NB)]],
                R_x.at[slot, pl.ds(0, _NB)], gsem).wait()
            pltpu.make_async_copy(
                ta_hbm.at[ixa_v.at[slot]],
                R_x.at[slot, pl.ds(_NB, _NA)], gsem).wait()

        build_ixa(0, 0)
        fire_gather(0, 0, gsem0)
        build_ixa(1, 1)

        def half_step(b, slot, o_slot, gsem, o_gsem, wsem):
            wait_gather(b, slot, gsem)
            @pl.when(b + 1 < _BPW)
            def _():
                fire_gather(b + 1, o_slot, o_gsem)
            @pl.when(b + 2 < _BPW)
            def _():
                build_ixa(b + 2, slot)
            @pl.when(b >= 2)
            def _():
                pltpu.make_async_copy(
                    h_v.at[slot],
                    h_hbm.at[pl.ds((base + b - 2) * _HROWS, _HROWS)],
                    wsem).wait()

            def f_loop(f, p):
                fbase = _NB + lax.shift_right_logical(f, 3) * _F
                flane = lax.bitwise_and(f, 7) * _D
                def g_loop(g, p):
                    va = R_x[slot, fbase + g,
                             pl.ds(pl.multiple_of(flane, 16), 16)]
                    vb = R_x[slot,
                             _NB + lax.shift_right_logical(g, 3) * _F + f,
                             pl.ds(pl.multiple_of(
                                 lax.bitwise_and(g, 7) * _D, 16), 16)]
                    h_v[slot, lax.shift_right_logical(p, 7),
                        pl.ds(pl.multiple_of(
                            lax.bitwise_and(p, 127), 16), 16)] = va * vb
                    return p + 16
                return lax.fori_loop(f + 1, 24, g_loop, p)
            p = lax.fori_loop(0, 24, f_loop, 0)

            for gB, blane in ((24, 0), (25, _D)):
                def fB_loop(f, p, gB=gB, blane=blane):
                    va = R_x[slot,
                             _NB + lax.shift_right_logical(f, 3) * _F + gB,
                             pl.ds(pl.multiple_of(
                                 lax.bitwise_and(f, 7) * _D, 16), 16)]
                    vb = R_x[slot, f, pl.ds(blane, 16)]
                    h_v[slot, lax.shift_right_logical(p, 7),
                        pl.ds(pl.multiple_of(
                            lax.bitwise_and(p, 127), 16), 16)] = va * vb
                    return p + 16
                p = lax.fori_loop(0, 24, fB_loop, p)

            va = R_x[slot, 25, pl.ds(0, 16)]
            vb = R_x[slot, 24, pl.ds(_D, 16)]
            h_v[slot, (_PAIRS - 1) // 8,
                pl.ds(((_PAIRS - 1) * 16) % 128, 16)] = va * vb

            def w_loop(g, acc):
                return acc + R_x[slot, g, pl.ds(32, 16)]
            h_v[slot, _HROWS - 1, pl.ds(80, 16)] = lax.fori_loop(
                0, _F, w_loop, jnp.zeros((16,), jnp.float32))

            pltpu.async_copy(
                h_v.at[slot],
                h_hbm.at[pl.ds((base + b) * _HROWS, _HROWS)], wsem)

        def iter_body(i, _):
            b0 = 2 * i
            half_step(b0, 0, 1, gsem0, gsem1, wsem0)
            half_step(b0 + 1, 1, 0, gsem1, gsem0, wsem1)
            return 0
        lax.fori_loop(0, _BPW // 2, iter_body, 0)

        pltpu.make_async_copy(
            h_v.at[0],
            h_hbm.at[pl.ds((base + _BPW - 2) * _HROWS, _HROWS)], wsem0).wait()
        pltpu.make_async_copy(
            h_v.at[1],
            h_hbm.at[pl.ds((base + _BPW - 1) * _HROWS, _HROWS)], wsem1).wait()

    return sc_interactions


_sc_interactions = _sc_make()


_BT = 512


def _mlp_body(h_ref, W1_ref, b1_ref, W2_ref, b2_ref, W3_ref, b3_ref, e_ref,
              out_ref):
    h2 = h_ref[...].reshape(_BT, _HROWS * 128)
    a1 = jnp.dot(h2, W1_ref[...], preferred_element_type=jnp.float32)
    a1 = jnp.maximum(a1 + b1_ref[...], 0.0)
    a2 = jnp.dot(a1, W2_ref[...], preferred_element_type=jnp.float32)
    a2 = jnp.maximum(a2 + b2_ref[...], 0.0)
    a3 = jnp.dot(a2, W3_ref[...], preferred_element_type=jnp.float32)
    fo = jnp.dot(h2, e_ref[...], preferred_element_type=jnp.float32)
    out_ref[...] = a3 + fo + b3_ref[...]


_mlp_call = pl.pallas_call(
    _mlp_body,
    grid=(_B // _BT,),
    in_specs=[
        pl.BlockSpec((_BT * _HROWS, 128), lambda i: (i, 0)),
        pl.BlockSpec((_HROWS * 128, 64), lambda i: (0, 0)),
        pl.BlockSpec((1, 64), lambda i: (0, 0)),
        pl.BlockSpec((64, 32), lambda i: (0, 0)),
        pl.BlockSpec((1, 32), lambda i: (0, 0)),
        pl.BlockSpec((32, 1), lambda i: (0, 0)),
        pl.BlockSpec((1, 1), lambda i: (0, 0)),
        pl.BlockSpec((_HROWS * 128, 1), lambda i: (0, 0)),
    ],
    out_specs=pl.BlockSpec((_BT, 1), lambda i: (i, 0)),
    out_shape=jax.ShapeDtypeStruct((_B, 1), jnp.float32),
)


def kernel(x, emb, w_lin, b_lin, W1, b1, W2, b2, W3, b3):
    x_off = x + jnp.asarray(_OFFS)[None, :]
    x_pad = jnp.pad(x_off, ((0, 0), (0, 128 - _F)))
    ta = jnp.concatenate(
        [jnp.transpose(emb[8 * p:8 * p + 8], (1, 0, 2)).reshape(_FEAT, 128)
         for p in range(3)], axis=0)
    tb = jnp.concatenate(
        [emb[24], emb[25], w_lin.reshape(_FEAT, 1),
         jnp.zeros((_FEAT, 128 - 2 * _D - 1), jnp.float32)], axis=1)
    (h,) = _sc_interactions(ta, tb, x_pad)
    W1perm = W1.reshape(_PAIRS, _D, 64)[_PERM].reshape(_INTER, 64)
    W1p = jnp.concatenate(
        [W1perm, jnp.zeros((_HROWS * 128 - _INTER, 64), jnp.float32)], axis=0)
    e = jnp.zeros((_HROWS * 128, 1), jnp.float32).at[_INTER, 0].set(1.0)
    out = _mlp_call(h, W1p, b1.reshape(1, 64), W2, b2.reshape(1, 32),
                    W3, (b3 + b_lin).reshape(1, 1), e)
    return out[:, 0]


# speedup vs baseline: 2.4420x; 2.0830x over previous
import functools

import numpy as np
import jax
import jax.numpy as jnp
from jax import lax
from jax.experimental import pallas as pl
from jax.experimental.pallas import tpu as pltpu
from jax.experimental.pallas import tpu_sc as plsc

_FIELD_DIMS = [1000] * 26
_F = 26
_FEAT = sum(_FIELD_DIMS)
_D = 16
_PAIRS = _F * (_F - 1) // 2
_INTER = _PAIRS * _D
_HROWS = 41
_B = 4096
_OFFS = np.asarray([0, *np.cumsum(_FIELD_DIMS)[:-1]], dtype=np.int32)

_NIDX = _F * _F
_NIDX_PAD = 688
_CHUNKS = (128, 128, 128, 128, 128, 48)

_NW = 32
_BPW = _B // _NW


def _pair_perm():
    def old_idx(f, g):
        f, g = min(f, g), max(f, g)
        return f * (2 * _F - f - 1) // 2 + (g - f - 1)
    perm = []
    for r in range(_F - 1):
        for m in range(13):
            if m == 0:
                a, b = r, _F - 1
            else:
                a = (r + m) % (_F - 1)
                b = (r - m) % (_F - 1)
            perm.append(old_idx(a, b))
    assert sorted(perm) == list(range(_PAIRS))
    return np.asarray(perm, dtype=np.int32)


_PERM = _pair_perm()


def _sc_make():
    mesh = plsc.VectorSubcoreMesh(core_axis_name="c", subcore_axis_name="s")

    @functools.partial(
        pl.kernel,
        mesh=mesh,
        compiler_params=pltpu.CompilerParams(
            use_tc_tiling_on_sc=False, needs_layout_passes=False),
        out_type=[
            jax.ShapeDtypeStruct((_B * _HROWS, 128), jnp.float32),
        ],
        scratch_types=[
            pltpu.VMEM((2, _NIDX_PAD), jnp.int32),
            pltpu.VMEM((_F, _BPW), jnp.int32),
            pltpu.VMEM((2, _NIDX_PAD, _D), jnp.float32),
            pltpu.VMEM((2, _HROWS, 128), jnp.float32),
            pltpu.VMEM((_F, _BPW, _D), jnp.float32),
            pltpu.SemaphoreType.DMA,
            pltpu.SemaphoreType.DMA,
            pltpu.SemaphoreType.DMA,
            pltpu.SemaphoreType.DMA,
        ],
    )
    def sc_interactions(emb_hbm, xoffT_hbm, wpad_hbm, h_hbm,
                        ix_v, idx_v, R_v, h_v, w_all,
                        gsem0, gsem1, wsem0, wsem1):
        wid = lax.axis_index("s") * 2 + lax.axis_index("c")
        base = wid * _BPW

        pltpu.sync_copy(xoffT_hbm.at[:, pl.ds(base, _BPW)], idx_v)
        for f in range(_F):
            pltpu.async_copy(wpad_hbm.at[idx_v.at[f]], w_all.at[f], wsem0)
        for f in range(_F):
            pltpu.make_async_copy(
                wpad_hbm.at[idx_v.at[f]], w_all.at[f], wsem0).wait()

        zero16 = jnp.zeros((16,), jnp.float32)
        for slot in (0, 1):
            for j in range(3):
                h_v[slot, _HROWS - 1, pl.ds(80 + j * 16, 16)] = zero16

        lane_iota = lax.iota(jnp.int32, 16)

        def build_ix(b, slot):
            bvec = jnp.zeros((16,), jnp.int32) + b
            def chunk(k, _):
                vj = lane_iota + k * 16
                vf = jnp.minimum(lax.div(vj, jnp.int32(_F)), _F - 1)
                vg = jnp.minimum(vj - vf * _F, _F - 1)
                xv = plsc.load_gather(idx_v, [vg, bvec])
                ix_v[slot, pl.ds(pl.multiple_of(k * 16, 16), 16)] = (
                    xv + vf * _FEAT)
                return 0
            lax.fori_loop(0, _NIDX_PAD // 16, chunk, 0)

        def fire_gather(slot, gsem):
            off = 0
            for sz in _CHUNKS:
                pltpu.async_copy(
                    emb_hbm.at[ix_v.at[slot, pl.ds(off, sz)]],
                    R_v.at[slot, pl.ds(off, sz)], gsem)
                off += sz

        def wait_gather(slot, gsem):
            off = 0
            for sz in _CHUNKS:
                pltpu.make_async_copy(
                    emb_hbm.at[ix_v.at[slot, pl.ds(off, sz)]],
                    R_v.at[slot, pl.ds(off, sz)], gsem).wait()
                off += sz

        build_ix(0, 0)
        fire_gather(0, gsem0)
        build_ix(1, 1)

        def half_step(b, slot, o_slot, gsem, o_gsem, wsem):
            wait_gather(slot, gsem)
            @pl.when(b + 1 < _BPW)
            def _():
                fire_gather(o_slot, o_gsem)
            @pl.when(b + 2 < _BPW)
            def _():
                build_ix(b + 2, slot)
            @pl.when(b >= 2)
            def _():
                pltpu.make_async_copy(
                    h_v.at[slot],
                    h_hbm.at[pl.ds((base + b - 2) * _HROWS, _HROWS)],
                    wsem).wait()

            def round_loop(r, _):
                p0 = r * (13 * 16)
                for m in range(13):
                    if m == 0:
                        a, bb = r, _F - 1
                    else:
                        a = r + m
                        a = jnp.where(a >= _F - 1, a - (_F - 1), a)
                        bb = r - m + (_F - 1)
                        bb = jnp.where(bb >= _F - 1, bb - (_F - 1), bb)
                    va = R_v[slot, a * _F + bb, :]
                    vb = R_v[slot, bb * _F + a, :]
                    p = p0 + m * 16
                    pr = lax.shift_right_logical(p, 7)
                    pc = lax.bitwise_and(p, 127)
                    h_v[slot, pr, pl.ds(pl.multiple_of(pc, 16), 16)] = va * vb
                return 0
            lax.fori_loop(0, _F - 1, round_loop, 0)

            def w_loop(f, acc):
                return acc + w_all[f, b, :]
            h_v[slot, _HROWS - 1, pl.ds(80, 16)] = lax.fori_loop(
                0, _F, w_loop, jnp.zeros((16,), jnp.float32))

            pltpu.async_copy(
                h_v.at[slot],
                h_hbm.at[pl.ds((base + b) * _HROWS, _HROWS)], wsem)

        def iter_body(i, _):
            b0 = 2 * i
            half_step(b0, 0, 1, gsem0, gsem1, wsem0)
            half_step(b0 + 1, 1, 0, gsem1, gsem0, wsem1)
            return 0
        lax.fori_loop(0, _BPW // 2, iter_body, 0)

        pltpu.make_async_copy(
            h_v.at[0],
            h_hbm.at[pl.ds((base + _BPW - 2) * _HROWS, _HROWS)], wsem0).wait()
        pltpu.make_async_copy(
            h_v.at[1],
            h_hbm.at[pl.ds((base + _BPW - 1) * _HROWS, _HROWS)], wsem1).wait()

    return sc_interactions


_sc_interactions = _sc_make()

_BT = 512


def _mlp_body(h_ref, W1_ref, b1_ref, W2_ref, b2_ref, W3_ref, b3_ref, e_ref,
              out_ref):
    h2 = h_ref[...].reshape(_BT, _HROWS * 128)
    a1 = jnp.dot(h2, W1_ref[...], preferred_element_type=jnp.float32)
    a1 = jnp.maximum(a1 + b1_ref[...], 0.0)
    a2 = jnp.dot(a1, W2_ref[...], preferred_element_type=jnp.float32)
    a2 = jnp.maximum(a2 + b2_ref[...], 0.0)
    a3 = jnp.dot(a2, W3_ref[...], preferred_element_type=jnp.float32)
    fo = jnp.dot(h2, e_ref[...], preferred_element_type=jnp.float32)
    out_ref[...] = a3 + fo + b3_ref[...]


_mlp_call = pl.pallas_call(
    _mlp_body,
    grid=(_B // _BT,),
    in_specs=[
        pl.BlockSpec((_BT * _HROWS, 128), lambda i: (i, 0)),
        pl.BlockSpec((_HROWS * 128, 64), lambda i: (0, 0)),
        pl.BlockSpec((1, 64), lambda i: (0, 0)),
        pl.BlockSpec((64, 32), lambda i: (0, 0)),
        pl.BlockSpec((1, 32), lambda i: (0, 0)),
        pl.BlockSpec((32, 1), lambda i: (0, 0)),
        pl.BlockSpec((1, 1), lambda i: (0, 0)),
        pl.BlockSpec((_HROWS * 128, 1), lambda i: (0, 0)),
    ],
    out_specs=pl.BlockSpec((_BT, 1), lambda i: (i, 0)),
    out_shape=jax.ShapeDtypeStruct((_B, 1), jnp.float32),
)


def kernel(x, emb, w_lin, b_lin, W1, b1, W2, b2, W3, b3):
    x_off = x + jnp.asarray(_OFFS)[None, :]
    emb_flat = emb.reshape(_F * _FEAT, _D)
    w_pad = jnp.concatenate(
        [w_lin.reshape(_FEAT, 1), jnp.zeros((_FEAT, 15), jnp.float32)], axis=1)
    (h,) = _sc_interactions(emb_flat, x_off.T, w_pad)
    W1perm = W1.reshape(_PAIRS, _D, 64)[_PERM].reshape(_INTER, 64)
    W1p = jnp.concatenate(
        [W1perm, jnp.zeros((_HROWS * 128 - _INTER, 64), jnp.float32)], axis=0)
    e = jnp.zeros((_HROWS * 128, 1), jnp.float32).at[_INTER, 0].set(1.0)
    out = _mlp_call(h, W1p, b1.reshape(1, 64), W2, b2.reshape(1, 32),
                    W3, (b3 + b_lin).reshape(1, 1), e)
    return out[:, 0]
